# baseline (speedup 1.0000x reference)
import functools

import jax
import jax.numpy as jnp
from jax import lax
from jax.experimental import pallas as pl
from jax.experimental.pallas import tpu as pltpu
from jax.experimental.pallas import tpu_sc as plsc

_NCODES = 16384
_BATCH = 4096
_D = 512
_L = 16
_NC = 2
_NS = 16
_NW = _NC * _NS
_BPW = _BATCH // _NW
_FCH = _BPW // _L
_NCHUNKS = _BATCH // _L


def _body(idx_hbm, data_hbm, out_hbm, idx_v, table_v, fsrc_v, fdst_v,
          rows_v, fbuf, csem, wsem, gsem):
    wid = lax.axis_index("s") * _NC + lax.axis_index("c")
    base = wid * _BPW

    lin_in = pltpu.async_copy(
        data_hbm.at[pl.ds(base, _BPW)], rows_v, csem)

    pltpu.sync_copy(idx_hbm, idx_v)

    lane = lax.iota(jnp.int32, _L)
    nxt_lane = (lane + 1) & (_L - 1)
    last_lane = lane == (_L - 1)

    def chunk_step(c, carry):
        chunk = idx_v[pl.ds(c * _L, _L)]
        comp = chunk * _L + lane
        sk, _ = plsc.sort_key_val(comp, comp)
        nxt = jnp.take(sk, nxt_lane, mode="wrap")
        code = sk >> 4
        is_last = jnp.logical_or(code != (nxt >> 4), last_lane)
        pos = (sk & (_L - 1)) + c * _L
        plsc.store_scatter(table_v, [code], pos, mask=is_last)
        return carry

    lax.fori_loop(0, _NCHUNKS, chunk_step, 0, unroll=8)

    my0 = idx_v[pl.ds(base, _L)]
    s0 = plsc.load_gather(table_v, [my0])
    zero = jnp.zeros((_L,), jnp.int32)
    pad_src = jnp.take(s0, zero, mode="wrap")
    pad_dst = zero + base
    for j in range(_FCH):
        fsrc_v[j, :] = pad_src
        fdst_v[j, :] = pad_dst

    n = jnp.int32(0)
    for b in range(_FCH):
        my = idx_v[pl.ds(base + b * _L, _L)]
        s = plsc.load_gather(table_v, [my])
        rows = base + b * _L + lane
        m = s != rows
        mi = m.astype(jnp.int32)
        posn = n + jnp.cumsum(mi) - 1
        plsc.store_scatter(fsrc_v, [posn >> 4, posn & (_L - 1)], s, mask=m)
        plsc.store_scatter(fdst_v, [posn >> 4, posn & (_L - 1)], rows,
                           mask=m)
        n = n + jnp.sum(mi)

    lin_in.wait()
    lin_out = pltpu.async_copy(
        rows_v, out_hbm.at[pl.ds(base, _BPW)], wsem)
    lin_out.wait()
    for j in range(_FCH):
        @pl.when(j * _L < n)
        def _patch():
            pltpu.async_copy(data_hbm.at[fsrc_v.at[j]], fbuf, gsem).wait()
            pltpu.async_copy(fbuf, out_hbm.at[fdst_v.at[j]], wsem).wait()


_cache_lookup = functools.partial(
    pl.kernel,
    out_type=jax.ShapeDtypeStruct((_BATCH, _D), jnp.float32),
    mesh=plsc.VectorSubcoreMesh(
        core_axis_name="c", subcore_axis_name="s",
        num_cores=_NC, num_subcores=_NS),
    scratch_types=[
        pltpu.VMEM((_BATCH,), jnp.int32),
        pltpu.VMEM((_NCODES,), jnp.int32),
        pltpu.VMEM((_FCH, _L), jnp.int32),
        pltpu.VMEM((_FCH, _L), jnp.int32),
        pltpu.VMEM((_BPW, _D), jnp.float32),
        pltpu.VMEM((_L, _D), jnp.float32),
        pltpu.SemaphoreType.DMA,
        pltpu.SemaphoreType.DMA,
        pltpu.SemaphoreType.DMA,
    ],
    compiler_params=pltpu.CompilerParams(needs_layout_passes=False),
)(_body)


@jax.jit
def kernel(seq_idxs, set_data, cache):
    del cache
    return _cache_lookup(seq_idxs.astype(jnp.int32), set_data)
